# Initial kernel scaffold; baseline (speedup 1.0000x reference)
import jax
import jax.numpy as jnp
from jax.experimental import pallas as pl


def kernel(x, edge_index, edge_attr, W1r, b1, W1s, W2r, b2, W2s, W3r, b3, W3s):
    raise NotImplementedError("write your pallas kernel here")


# speedup vs baseline: 4.8776x; 4.8776x over previous
import functools

import jax
import jax.numpy as jnp
from jax import lax
from jax.experimental import pallas as pl
from jax.experimental.pallas import tpu as pltpu
from jax.experimental.pallas import tpu_sc as plsc

_N = 10000
_E = 320000
_D = 128
_H = 64

_NC = 2
_NS = 16
_NW = _NC * _NS
_EPT = _E // _NW
_CH = 80
_NCHUNK = _EPT // _CH
_NPAD = 10240
_RPT = _NPAD // _NS

_GATHER_DNUMS = lax.GatherDimensionNumbers(
    offset_dims=(), collapsed_slice_dims=(0,), start_index_map=(0,))


def _splat_lane(vec, lane):
    idx = jnp.full((16, 1), lane, jnp.int32)
    return lax.gather(vec, idx, _GATHER_DNUMS, (1,),
                      mode=lax.GatherScatterMode.PROMISE_IN_BOUNDS)


@functools.cache
def _make_sc_segment():
    mesh = plsc.VectorSubcoreMesh(core_axis_name="c", subcore_axis_name="s")
    return pl.kernel(
        _sc_segment_body,
        out_type=jax.ShapeDtypeStruct((_NC, _NS, _RPT, _H), jnp.float32),
        mesh=mesh,
        scratch_types=[
            pltpu.VMEM((_NCHUNK, _CH), jnp.int32),
            pltpu.VMEM((_NCHUNK, _CH), jnp.int32),
            pltpu.VMEM((_NCHUNK, _CH), jnp.float32),
            pltpu.VMEM((_CH, _H), jnp.float32),
            pltpu.VMEM_SHARED((_NPAD, _H), jnp.float32),
            pltpu.SemaphoreType.DMA,
        ],
        compiler_params=pltpu.CompilerParams(use_tc_tiling_on_sc=False),
    )


def _sc_segment(*args):
    return _make_sc_segment()(*args)


def _sc_segment_body(hr, src_h, dst_h, w_h, zeros_h, out,
                     src_v, dst_v, w_v, rows_v, acc, sem):
    cid = lax.axis_index("c")
    sid = lax.axis_index("s")
    wid = sid * _NC + cid

    pltpu.sync_copy(src_h.at[wid], src_v)
    pltpu.sync_copy(dst_h.at[wid], dst_v)
    pltpu.sync_copy(w_h.at[wid], w_v)

    pltpu.sync_copy(zeros_h, acc.at[pl.ds(sid * _RPT, _RPT)])
    plsc.subcore_barrier()

    def chunk_body(j, carry):
        pltpu.async_copy(hr.at[src_v.at[j]], rows_v, sem).wait()

        def group_body(g, c):
            w16 = w_v[j, pl.ds(g * 16, 16)]
            for e16 in range(16):
                e = g * 16 + e16
                wsplat = _splat_lane(w16, e16)
                for f in range(_H // 16):
                    rows_v[e, pl.ds(f * 16, 16)] = (
                        rows_v[e, pl.ds(f * 16, 16)] * wsplat)
            return c

        lax.fori_loop(0, _CH // 16, group_body, 0)

        pltpu.sync_copy(rows_v, acc.at[dst_v.at[j]], add=True)
        return carry

    lax.fori_loop(0, _NCHUNK, chunk_body, 0)
    plsc.subcore_barrier()

    pltpu.sync_copy(acc.at[pl.ds(sid * _RPT, _RPT)], out.at[cid, sid])


def _dot_t(a, w):
    return lax.dot_general(a, w, (((1,), (1,)), ((), ())),
                           preferred_element_type=jnp.float32)


def _pre_body(x_ref, wr_ref, ws_ref, hr_ref, hs_ref):
    x = x_ref[...]
    hr_ref[...] = _dot_t(x, wr_ref[...])
    hs_ref[...] = _dot_t(x, ws_ref[...])


def _mid_body(p_ref, hs_ref, b_ref, wr_ref, ws_ref, hr_ref, hs2_ref):
    h = jnp.maximum(p_ref[0] + p_ref[1] + hs_ref[...] + b_ref[...], 0.0)
    hr_ref[...] = _dot_t(h, wr_ref[...])
    hs2_ref[...] = _dot_t(h, ws_ref[...])


def _relu_body(p_ref, hs_ref, b_ref, h_ref):
    h_ref[...] = jnp.maximum(p_ref[0] + p_ref[1] + hs_ref[...] + b_ref[...],
                             0.0)


def _final_body(p_ref, h_ref, b_ref, wr_ref, ws_ref, out_ref):
    agg = p_ref[0] + p_ref[1]
    out_ref[...] = (_dot_t(agg, wr_ref[...]) + b_ref[...]
                    + _dot_t(h_ref[...], ws_ref[...]))


def _f32(*shape):
    return jax.ShapeDtypeStruct(shape, jnp.float32)


def kernel(x, edge_index, edge_attr, W1r, b1, W1s, W2r, b2, W2s, W3r, b3, W3s):
    src = edge_index[0].reshape(_NW, _NCHUNK, _CH)
    dst = edge_index[1].reshape(_NW, _NCHUNK, _CH)
    w = edge_attr.reshape(_NW, _NCHUNK, _CH)
    zeros = jnp.zeros((_RPT, _H), jnp.float32)

    hr1, hs1 = pl.pallas_call(
        _pre_body, out_shape=[_f32(_N, _H), _f32(_N, _H)])(x, W1r, W1s)
    p1 = _sc_segment(hr1, src, dst, w, zeros).reshape(_NC, _NPAD, _H)[:, :_N]

    hr2, hs2 = pl.pallas_call(
        _mid_body, out_shape=[_f32(_N, _H), _f32(_N, _H)])(
            p1, hs1, b1.reshape(1, _H), W2r, W2s)
    p2 = _sc_segment(hr2, src, dst, w, zeros).reshape(_NC, _NPAD, _H)[:, :_N]

    h2 = pl.pallas_call(
        _relu_body, out_shape=_f32(_N, _H))(p2, hs2, b2.reshape(1, _H))
    p3 = _sc_segment(h2, src, dst, w, zeros).reshape(_NC, _NPAD, _H)[:, :_N]

    out = pl.pallas_call(
        _final_body, out_shape=_f32(_N, _D))(
            p3, h2, b3.reshape(1, _D), W3r, W3s)
    return out


# speedup vs baseline: 8.6639x; 1.7763x over previous
import functools

import jax
import jax.numpy as jnp
from jax import lax
from jax.experimental import pallas as pl
from jax.experimental.pallas import tpu as pltpu
from jax.experimental.pallas import tpu_sc as plsc

_N = 10000
_E = 320000
_D = 128
_H = 64

_NC = 2
_NS = 16
_NW = _NC * _NS
_EPT = _E // _NW
_CH = 80
_NCHUNK = _EPT // _CH
_NPAD = 10240
_RPT = _NPAD // _NS

_GATHER_DNUMS = lax.GatherDimensionNumbers(
    offset_dims=(), collapsed_slice_dims=(0,), start_index_map=(0,))


def _splat_lane(vec, lane):
    idx = jnp.full((16, 1), lane, jnp.int32)
    return lax.gather(vec, idx, _GATHER_DNUMS, (1,),
                      mode=lax.GatherScatterMode.PROMISE_IN_BOUNDS)


_NBUF = 5
_NOUT = _NCHUNK // _NBUF


@functools.cache
def _make_sc_segment():
    mesh = plsc.VectorSubcoreMesh(core_axis_name="c", subcore_axis_name="s")
    return pl.kernel(
        _sc_segment_body,
        out_type=jax.ShapeDtypeStruct((_NC, _NS, _RPT, _H), jnp.float32),
        mesh=mesh,
        scratch_types=[
            pltpu.VMEM((_NCHUNK, _CH), jnp.int32),
            pltpu.VMEM((_NCHUNK, _CH), jnp.int32),
            pltpu.VMEM((_NCHUNK, _CH), jnp.float32),
            pltpu.VMEM((_NBUF, _CH, _H), jnp.float32),
            pltpu.VMEM_SHARED((_NPAD, _H), jnp.float32),
        ] + [pltpu.SemaphoreType.DMA] * (2 * _NBUF),
        compiler_params=pltpu.CompilerParams(use_tc_tiling_on_sc=False),
    )


def _sc_segment(*args):
    return _make_sc_segment()(*args)


def _sc_segment_body(hr, src_h, dst_h, w_h, zeros_h, out,
                     src_v, dst_v, w_v, rows_v, acc, *sems):
    gsem = sems[:_NBUF]
    ssem = sems[_NBUF:]
    cid = lax.axis_index("c")
    sid = lax.axis_index("s")
    wid = sid * _NC + cid

    pltpu.sync_copy(src_h.at[wid], src_v)
    pltpu.sync_copy(dst_h.at[wid], dst_v)
    pltpu.sync_copy(w_h.at[wid], w_v)

    pltpu.sync_copy(zeros_h, acc.at[pl.ds(sid * _RPT, _RPT)])
    plsc.subcore_barrier()

    for b in range(_NBUF - 1):
        pltpu.async_copy(hr.at[src_v.at[b]], rows_v.at[b], gsem[b])

    def scale(j, b):
        def group_body(g, c):
            w16 = w_v[j, pl.ds(g * 16, 16)]
            for e16 in range(16):
                e = g * 16 + e16
                wsplat = _splat_lane(w16, e16)
                for f in range(_H // 16):
                    rows_v[b, e, pl.ds(f * 16, 16)] = (
                        rows_v[b, e, pl.ds(f * 16, 16)] * wsplat)
            return c

        lax.fori_loop(0, _CH // 16, group_body, 0)

    def outer_body(go, carry):
        for b in range(_NBUF):
            j = go * _NBUF + b
            pltpu.make_async_copy(
                hr.at[pl.ds(0, _CH)], rows_v.at[b], gsem[b]).wait()
            scale(j, b)
            pltpu.async_copy(rows_v.at[b], acc.at[dst_v.at[j]], ssem[b],
                             add=True)
            bn = (b + _NBUF - 1) % _NBUF
            jn = j + _NBUF - 1

            @pl.when(jn < _NCHUNK)
            def _refill():
                @pl.when(j >= 1)
                def _drain():
                    pltpu.make_async_copy(
                        rows_v.at[bn], acc.at[dst_v.at[0]], ssem[bn]).wait()
                pltpu.async_copy(hr.at[src_v.at[jn]], rows_v.at[bn], gsem[bn])
        return carry

    lax.fori_loop(0, _NOUT, outer_body, 0)
    for b in range(_NBUF):
        pltpu.make_async_copy(
            rows_v.at[b], acc.at[dst_v.at[0]], ssem[b]).wait()
    plsc.subcore_barrier()

    pltpu.sync_copy(acc.at[pl.ds(sid * _RPT, _RPT)], out.at[cid, sid])


def _dot_t(a, w):
    return lax.dot_general(a, w, (((1,), (1,)), ((), ())),
                           preferred_element_type=jnp.float32)


def _pre_body(x_ref, wr_ref, ws_ref, hr_ref, hs_ref):
    x = x_ref[...]
    hr_ref[...] = _dot_t(x, wr_ref[...])
    hs_ref[...] = _dot_t(x, ws_ref[...])


def _mid_body(p_ref, hs_ref, b_ref, wr_ref, ws_ref, hr_ref, hs2_ref):
    h = jnp.maximum(p_ref[0] + p_ref[1] + hs_ref[...] + b_ref[...], 0.0)
    hr_ref[...] = _dot_t(h, wr_ref[...])
    hs2_ref[...] = _dot_t(h, ws_ref[...])


def _relu_body(p_ref, hs_ref, b_ref, h_ref):
    h_ref[...] = jnp.maximum(p_ref[0] + p_ref[1] + hs_ref[...] + b_ref[...],
                             0.0)


def _final_body(p_ref, h_ref, b_ref, wr_ref, ws_ref, out_ref):
    agg = p_ref[0] + p_ref[1]
    out_ref[...] = (_dot_t(agg, wr_ref[...]) + b_ref[...]
                    + _dot_t(h_ref[...], ws_ref[...]))


def _f32(*shape):
    return jax.ShapeDtypeStruct(shape, jnp.float32)


def kernel(x, edge_index, edge_attr, W1r, b1, W1s, W2r, b2, W2s, W3r, b3, W3s):
    src = edge_index[0].reshape(_NW, _NCHUNK, _CH)
    dst = edge_index[1].reshape(_NW, _NCHUNK, _CH)
    w = edge_attr.reshape(_NW, _NCHUNK, _CH)
    zeros = jnp.zeros((_RPT, _H), jnp.float32)

    hr1, hs1 = pl.pallas_call(
        _pre_body, out_shape=[_f32(_N, _H), _f32(_N, _H)])(x, W1r, W1s)
    p1 = _sc_segment(hr1, src, dst, w, zeros).reshape(_NC, _NPAD, _H)[:, :_N]

    hr2, hs2 = pl.pallas_call(
        _mid_body, out_shape=[_f32(_N, _H), _f32(_N, _H)])(
            p1, hs1, b1.reshape(1, _H), W2r, W2s)
    p2 = _sc_segment(hr2, src, dst, w, zeros).reshape(_NC, _NPAD, _H)[:, :_N]

    h2 = pl.pallas_call(
        _relu_body, out_shape=_f32(_N, _H))(p2, hs2, b2.reshape(1, _H))
    p3 = _sc_segment(h2, src, dst, w, zeros).reshape(_NC, _NPAD, _H)[:, :_N]

    out = pl.pallas_call(
        _final_body, out_shape=_f32(_N, _D))(
            p3, h2, b3.reshape(1, _D), W3r, W3s)
    return out


# speedup vs baseline: 15.6096x; 1.8017x over previous
import functools

import jax
import jax.numpy as jnp
from jax import lax
from jax.experimental import pallas as pl
from jax.experimental.pallas import tpu as pltpu
from jax.experimental.pallas import tpu_sc as plsc

_N = 10000
_E = 320000
_D = 128
_H = 64

_NC = 2
_NS = 16
_NW = _NC * _NS
_EPT = _E // _NW
_CH = 80
_NCHUNK = _EPT // _CH
_NPAD = 10240
_RPT = _NPAD // _NS

_GATHER_DNUMS = lax.GatherDimensionNumbers(
    offset_dims=(), collapsed_slice_dims=(0,), start_index_map=(0,))


def _splat_lane(vec, lane):
    idx = jnp.full((16, 1), lane, jnp.int32)
    return lax.gather(vec, idx, _GATHER_DNUMS, (1,),
                      mode=lax.GatherScatterMode.PROMISE_IN_BOUNDS)


_NBUF = 5
_NOUT = _NCHUNK // _NBUF


@functools.cache
def _make_sc_segment():
    mesh = plsc.VectorSubcoreMesh(core_axis_name="c", subcore_axis_name="s")
    return pl.kernel(
        _sc_segment_body,
        out_type=jax.ShapeDtypeStruct((_NC, _NS, _RPT, _H), jnp.float32),
        mesh=mesh,
        scratch_types=[
            pltpu.VMEM((_NCHUNK, _CH), jnp.int32),
            pltpu.VMEM((_NCHUNK, _CH), jnp.int32),
            pltpu.VMEM((_NCHUNK, _CH), jnp.float32),
            pltpu.VMEM((_NBUF, _CH, _H), jnp.float32),
            pltpu.VMEM((_NBUF, _CH, _H), jnp.float32),
            pltpu.VMEM_SHARED((_NPAD, _H), jnp.float32),
        ] + [pltpu.SemaphoreType.DMA] * (2 * _NBUF),
        compiler_params=pltpu.CompilerParams(use_tc_tiling_on_sc=False),
    )


def _sc_segment(*args):
    return _make_sc_segment()(*args)


def _sc_segment_body(hr, src_h, dst_h, w_h, zeros_h, out,
                     src_v, dst_v, w_v, rows_v, srows_v, acc, *sems):
    gsem = sems[:_NBUF]
    ssem = sems[_NBUF:]
    cid = lax.axis_index("c")
    sid = lax.axis_index("s")
    wid = sid * _NC + cid

    pltpu.sync_copy(src_h.at[wid], src_v)
    pltpu.sync_copy(dst_h.at[wid], dst_v)
    pltpu.sync_copy(w_h.at[wid], w_v)

    pltpu.sync_copy(zeros_h, acc.at[pl.ds(sid * _RPT, _RPT)])
    plsc.subcore_barrier()

    for b in range(_NBUF):
        pltpu.async_copy(hr.at[src_v.at[b]], rows_v.at[b], gsem[b])

    def scale(j, b):
        def group_body(g, c):
            w16 = w_v[j, pl.ds(g * 16, 16)]
            for e16 in range(16):
                e = g * 16 + e16
                wsplat = _splat_lane(w16, e16)
                for f in range(_H // 16):
                    srows_v[b, e, pl.ds(f * 16, 16)] = (
                        rows_v[b, e, pl.ds(f * 16, 16)] * wsplat)
            return c

        lax.fori_loop(0, _CH // 16, group_body, 0)

    def outer_body(go, carry):
        for b in range(_NBUF):
            j = go * _NBUF + b
            pltpu.make_async_copy(
                hr.at[pl.ds(0, _CH)], rows_v.at[b], gsem[b]).wait()
            @pl.when(j >= _NBUF)
            def _drain():
                pltpu.make_async_copy(
                    srows_v.at[b], acc.at[dst_v.at[0]], ssem[b]).wait()
            scale(j, b)
            pltpu.async_copy(srows_v.at[b], acc.at[dst_v.at[j]], ssem[b],
                             add=True)
            jn = j + _NBUF

            @pl.when(jn < _NCHUNK)
            def _refill():
                pltpu.async_copy(hr.at[src_v.at[jn]], rows_v.at[b], gsem[b])
        return carry

    lax.fori_loop(0, _NOUT, outer_body, 0)
    for b in range(_NBUF):
        pltpu.make_async_copy(
            srows_v.at[b], acc.at[dst_v.at[0]], ssem[b]).wait()
    plsc.subcore_barrier()

    pltpu.sync_copy(acc.at[pl.ds(sid * _RPT, _RPT)], out.at[cid, sid])


def _dot_t(a, w):
    return lax.dot_general(a, w, (((1,), (1,)), ((), ())),
                           preferred_element_type=jnp.float32)


def _pre_body(x_ref, wr_ref, ws_ref, hr_ref, hs_ref):
    x = x_ref[...]
    hr_ref[...] = _dot_t(x, wr_ref[...])
    hs_ref[...] = _dot_t(x, ws_ref[...])


def _mid_body(p_ref, hs_ref, b_ref, wr_ref, ws_ref, hr_ref, hs2_ref):
    h = jnp.maximum(p_ref[0] + p_ref[1] + hs_ref[...] + b_ref[...], 0.0)
    hr_ref[...] = _dot_t(h, wr_ref[...])
    hs2_ref[...] = _dot_t(h, ws_ref[...])


def _relu_body(p_ref, hs_ref, b_ref, h_ref):
    h_ref[...] = jnp.maximum(p_ref[0] + p_ref[1] + hs_ref[...] + b_ref[...],
                             0.0)


def _final_body(p_ref, h_ref, b_ref, wr_ref, ws_ref, out_ref):
    agg = p_ref[0] + p_ref[1]
    out_ref[...] = (_dot_t(agg, wr_ref[...]) + b_ref[...]
                    + _dot_t(h_ref[...], ws_ref[...]))


def _f32(*shape):
    return jax.ShapeDtypeStruct(shape, jnp.float32)


def kernel(x, edge_index, edge_attr, W1r, b1, W1s, W2r, b2, W2s, W3r, b3, W3s):
    src = edge_index[0].reshape(_NW, _NCHUNK, _CH)
    dst = edge_index[1].reshape(_NW, _NCHUNK, _CH)
    w = edge_attr.reshape(_NW, _NCHUNK, _CH)
    zeros = jnp.zeros((_RPT, _H), jnp.float32)

    hr1, hs1 = pl.pallas_call(
        _pre_body, out_shape=[_f32(_N, _H), _f32(_N, _H)])(x, W1r, W1s)
    p1 = _sc_segment(hr1, src, dst, w, zeros).reshape(_NC, _NPAD, _H)[:, :_N]

    hr2, hs2 = pl.pallas_call(
        _mid_body, out_shape=[_f32(_N, _H), _f32(_N, _H)])(
            p1, hs1, b1.reshape(1, _H), W2r, W2s)
    p2 = _sc_segment(hr2, src, dst, w, zeros).reshape(_NC, _NPAD, _H)[:, :_N]

    h2 = pl.pallas_call(
        _relu_body, out_shape=_f32(_N, _H))(p2, hs2, b2.reshape(1, _H))
    p3 = _sc_segment(h2, src, dst, w, zeros).reshape(_NC, _NPAD, _H)[:, :_N]

    out = pl.pallas_call(
        _final_body, out_shape=_f32(_N, _D))(
            p3, h2, b3.reshape(1, _D), W3r, W3s)
    return out


# speedup vs baseline: 16.6538x; 1.0669x over previous
import functools

import jax
import jax.numpy as jnp
from jax import lax
from jax.experimental import pallas as pl
from jax.experimental.pallas import tpu as pltpu
from jax.experimental.pallas import tpu_sc as plsc

_N = 10000
_E = 320000
_D = 128
_H = 64

_NC = 2
_NS = 16
_NW = _NC * _NS
_EPT = _E // _NW
_CH = 80
_NCHUNK = _EPT // _CH
_NPAD = 10240
_RPT = _NPAD // _NS

_GATHER_DNUMS = lax.GatherDimensionNumbers(
    offset_dims=(), collapsed_slice_dims=(0,), start_index_map=(0,))


def _splat_lane(vec, lane):
    idx = jnp.full((16, 1), lane, jnp.int32)
    return lax.gather(vec, idx, _GATHER_DNUMS, (1,),
                      mode=lax.GatherScatterMode.PROMISE_IN_BOUNDS)


_NBUF = 5
_NOUT = _NCHUNK // _NBUF


@functools.cache
def _make_sc_segment():
    mesh = plsc.VectorSubcoreMesh(core_axis_name="c", subcore_axis_name="s")
    return pl.kernel(
        _sc_segment_body,
        out_type=jax.ShapeDtypeStruct((_NC, _NS, _RPT, _H), jnp.float32),
        mesh=mesh,
        scratch_types=[
            pltpu.VMEM((_NCHUNK, _CH), jnp.int32),
            pltpu.VMEM((_NCHUNK, _CH), jnp.int32),
            pltpu.VMEM((_NCHUNK, _CH), jnp.float32),
            pltpu.VMEM((_NBUF, _CH, _H), jnp.float32),
            pltpu.VMEM((_NBUF, _CH, _H), jnp.float32),
            pltpu.VMEM_SHARED((_NPAD, _H), jnp.float32),
        ] + [pltpu.SemaphoreType.DMA] * (2 * _NBUF),
        compiler_params=pltpu.CompilerParams(use_tc_tiling_on_sc=False),
    )


def _sc_segment(*args):
    return _make_sc_segment()(*args)


def _sc_segment_body(hr, src_h, dst_h, w_h, zeros_h, out,
                     src_v, dst_v, w_v, rows_v, srows_v, acc, *sems):
    gsem = sems[:_NBUF]
    ssem = sems[_NBUF:]
    cid = lax.axis_index("c")
    sid = lax.axis_index("s")
    wid = sid * _NC + cid

    pltpu.sync_copy(src_h.at[wid], src_v)
    pltpu.sync_copy(dst_h.at[wid], dst_v)
    pltpu.sync_copy(w_h.at[wid], w_v)

    pltpu.sync_copy(zeros_h, acc.at[pl.ds(sid * _RPT, _RPT)])
    plsc.subcore_barrier()

    for b in range(_NBUF):
        pltpu.async_copy(hr.at[src_v.at[b]], rows_v.at[b], gsem[b])

    def scale(j, b):
        def group_body(g, c):
            w16 = w_v[j, pl.ds(g * 16, 16)]
            for e16 in range(16):
                e = g * 16 + e16
                wsplat = _splat_lane(w16, e16)
                for f in range(_H // 16):
                    srows_v[b, e, pl.ds(f * 16, 16)] = (
                        rows_v[b, e, pl.ds(f * 16, 16)] * wsplat)
            return c

        lax.fori_loop(0, _CH // 16, group_body, 0)

    def outer_body(go, carry):
        for b in range(_NBUF):
            j = go * _NBUF + b
            pltpu.make_async_copy(
                hr.at[pl.ds(0, _CH)], rows_v.at[b], gsem[b]).wait()
            @pl.when(j >= _NBUF)
            def _drain():
                pltpu.make_async_copy(
                    srows_v.at[b], acc.at[dst_v.at[0]], ssem[b]).wait()
            scale(j, b)
            pltpu.async_copy(srows_v.at[b], acc.at[dst_v.at[j]], ssem[b],
                             add=True)
            jn = j + _NBUF

            @pl.when(jn < _NCHUNK)
            def _refill():
                pltpu.async_copy(hr.at[src_v.at[jn]], rows_v.at[b], gsem[b])
        return carry

    lax.fori_loop(0, _NOUT, outer_body, 0)
    for b in range(_NBUF):
        pltpu.make_async_copy(
            srows_v.at[b], acc.at[dst_v.at[0]], ssem[b]).wait()
    plsc.subcore_barrier()

    pltpu.sync_copy(acc.at[pl.ds(sid * _RPT, _RPT)], out.at[cid, sid])


def _dot_t(a, w):
    return lax.dot_general(a, w, (((1,), (1,)), ((), ())),
                           preferred_element_type=jnp.float32)


def _pre_body(x_ref, wr_ref, ws_ref, hr_ref, hs_ref):
    x = x_ref[...]
    hr_ref[...] = _dot_t(x, wr_ref[...])
    hs_ref[...] = _dot_t(x, ws_ref[...])


def _mid_body(p_ref, hs_ref, b_ref, wr_ref, ws_ref, hr_ref, hs2_ref):
    h = jnp.maximum(p_ref[0, :_N] + p_ref[1, :_N] + hs_ref[...] + b_ref[...],
                    0.0)
    hr_ref[...] = _dot_t(h, wr_ref[...])
    hs2_ref[...] = _dot_t(h, ws_ref[...])


def _relu_body(p_ref, hs_ref, b_ref, h_ref):
    h_ref[...] = jnp.maximum(p_ref[0, :_N] + p_ref[1, :_N] + hs_ref[...]
                             + b_ref[...], 0.0)


def _final_body(p_ref, h_ref, b_ref, wr_ref, ws_ref, out_ref):
    agg = p_ref[0, :_N] + p_ref[1, :_N]
    out_ref[...] = (_dot_t(agg, wr_ref[...]) + b_ref[...]
                    + _dot_t(h_ref[...], ws_ref[...]))


def _f32(*shape):
    return jax.ShapeDtypeStruct(shape, jnp.float32)


def kernel(x, edge_index, edge_attr, W1r, b1, W1s, W2r, b2, W2s, W3r, b3, W3s):
    src = edge_index[0].reshape(_NW, _NCHUNK, _CH)
    dst = edge_index[1].reshape(_NW, _NCHUNK, _CH)
    w = edge_attr.reshape(_NW, _NCHUNK, _CH)
    zeros = jnp.zeros((_RPT, _H), jnp.float32)

    hr1, hs1 = pl.pallas_call(
        _pre_body, out_shape=[_f32(_N, _H), _f32(_N, _H)])(x, W1r, W1s)
    p1 = _sc_segment(hr1, src, dst, w, zeros).reshape(_NC, _NPAD, _H)

    hr2, hs2 = pl.pallas_call(
        _mid_body, out_shape=[_f32(_N, _H), _f32(_N, _H)])(
            p1, hs1, b1.reshape(1, _H), W2r, W2s)
    p2 = _sc_segment(hr2, src, dst, w, zeros).reshape(_NC, _NPAD, _H)

    h2 = pl.pallas_call(
        _relu_body, out_shape=_f32(_N, _H))(p2, hs2, b2.reshape(1, _H))
    p3 = _sc_segment(h2, src, dst, w, zeros).reshape(_NC, _NPAD, _H)

    out = pl.pallas_call(
        _final_body, out_shape=_f32(_N, _D))(
            p3, h2, b3.reshape(1, _D), W3r, W3s)
    return out
